# baseline (speedup 1.0000x reference)
import jax
import jax.numpy as jnp
from jax.experimental import pallas as pl
from jax.experimental.pallas import tpu as pltpu
from jax.experimental.pallas import tpu_sc as plsc

_K = 16
_TC = 8192


def _linearize_table(weight):
    rows, dim = weight.shape
    out_rows = rows * dim // 128
    blk_out = _TC * dim // 128
    grid = (rows + _TC - 1) // _TC

    group = 128 // dim

    def tbody(i_ref, o_ref, s_ref):
        s_ref[...] = i_ref[...].T
        for a in range(group):
            o_ref[:, dim * a : dim * (a + 1)] = s_ref[a::group, :]

    return pl.pallas_call(
        tbody,
        grid=(grid,),
        in_specs=[pl.BlockSpec((dim, _TC), lambda i: (0, i))],
        out_specs=pl.BlockSpec((blk_out, 128), lambda i: (i, 0)),
        out_shape=jax.ShapeDtypeStruct((out_rows, 128), weight.dtype),
        scratch_shapes=[pltpu.VMEM((_TC, dim), weight.dtype)],
    )(weight.T)


def _gather(weight, token_ids):
    batch, hist = token_ids.shape
    dim = weight.shape[1]
    mesh = plsc.VectorSubcoreMesh(core_axis_name="c", subcore_axis_name="s")

    @pl.kernel(
        out_type=jax.ShapeDtypeStruct((batch, hist, dim), weight.dtype),
        mesh=mesh,
        scratch_types=[pltpu.SemaphoreType.DMA],
        compiler_params=pltpu.CompilerParams(use_tc_tiling_on_sc=False),
    )
    def gather_kernel(w_hbm, i_hbm, o_hbm, sem):
        def body(i_vmem, o_vmem):
            copies = [
                pltpu.async_copy(
                    w_hbm.at[i_vmem.at[j]],
                    o_vmem.at[j],
                    sem,
                )
                for j in range(_K)
            ]
            for c in copies:
                c.wait()

        pltpu.emit_pipeline(
            body,
            grid=(batch // _K,),
            in_specs=[pl.BlockSpec((_K, hist), index_map=lambda i: (i, 0))],
            out_specs=[
                pl.BlockSpec((_K, hist, dim), index_map=lambda i: (i, 0, 0))
            ],
            core_axis_name=("c", "s"),
            dimension_semantics=(pltpu.PARALLEL,),
        )(i_hbm, o_hbm)

    return gather_kernel(weight, token_ids)


def kernel(token_ids, weight):
    hist = token_ids.shape[1]
    weight = _linearize_table(weight).reshape(weight.shape)
    half = hist // 2
    return jnp.concatenate(
        [
            _gather(weight, token_ids[:, :half]),
            _gather(weight, token_ids[:, half:]),
        ],
        axis=1,
    )


# speedup vs baseline: 1.0229x; 1.0229x over previous
import jax
import jax.numpy as jnp
from jax.experimental import pallas as pl
from jax.experimental.pallas import tpu as pltpu
from jax.experimental.pallas import tpu_sc as plsc

_K = 16
_TC = 16384


def _linearize_table(weight):
    rows, dim = weight.shape
    out_rows = rows * dim // 128
    blk_out = _TC * dim // 128
    grid = (rows + _TC - 1) // _TC

    group = 128 // dim

    def tbody(i_ref, o_ref, s_ref):
        s_ref[...] = i_ref[...].T
        for a in range(group):
            o_ref[:, dim * a : dim * (a + 1)] = s_ref[a::group, :]

    return pl.pallas_call(
        tbody,
        grid=(grid,),
        in_specs=[pl.BlockSpec((dim, _TC), lambda i: (0, i))],
        out_specs=pl.BlockSpec((blk_out, 128), lambda i: (i, 0)),
        out_shape=jax.ShapeDtypeStruct((out_rows, 128), weight.dtype),
        scratch_shapes=[pltpu.VMEM((_TC, dim), weight.dtype)],
    )(weight.T)


def _gather(weight, token_ids):
    batch, hist = token_ids.shape
    dim = weight.shape[1]
    mesh = plsc.VectorSubcoreMesh(core_axis_name="c", subcore_axis_name="s")

    @pl.kernel(
        out_type=jax.ShapeDtypeStruct((batch, hist, dim), weight.dtype),
        mesh=mesh,
        scratch_types=[pltpu.SemaphoreType.DMA],
        compiler_params=pltpu.CompilerParams(use_tc_tiling_on_sc=False),
    )
    def gather_kernel(w_hbm, i_hbm, o_hbm, sem):
        def body(i_vmem, o_vmem):
            copies = [
                pltpu.async_copy(
                    w_hbm.at[i_vmem.at[j]],
                    o_vmem.at[j],
                    sem,
                )
                for j in range(_K)
            ]
            for c in copies:
                c.wait()

        pltpu.emit_pipeline(
            body,
            grid=(batch // _K,),
            in_specs=[pl.BlockSpec((_K, hist), index_map=lambda i: (i, 0))],
            out_specs=[
                pl.BlockSpec((_K, hist, dim), index_map=lambda i: (i, 0, 0))
            ],
            core_axis_name=("c", "s"),
            dimension_semantics=(pltpu.PARALLEL,),
        )(i_hbm, o_hbm)

    return gather_kernel(weight, token_ids)


def kernel(token_ids, weight):
    weight_lin = _linearize_table(weight).reshape(weight.shape)
    return _gather(weight_lin, token_ids)
